# baseline (device time: 79835 ns/iter reference)
import jax
import jax.numpy as jnp
from jax import lax
from jax.experimental import pallas as pl
from jax.experimental.pallas import tpu as pltpu

N_DEV = 4
B = 4
SQ = 256
SKV = 1024
H_LOC = 8
DH = 128
D = 1024
SCALE = 0.08838834764831843

P = 2
R = SQ // P
N_SLOT = P * B


def kernel(x, Wq, Wo, K_ext, V_ext):
    def body(x_ref, wq_ref, wo_ref, k_hbm, v_hbm, out_ref,
             part_ref, rs_in, attn_ref, kbuf, vbuf,
             ksems, vsems, rs_send, rs_recv, ag_send, ag_recv):
        d = lax.axis_index("i")
        left = lax.rem(d + N_DEV - 1, N_DEV)
        right = lax.rem(d + 1, N_DEV)
        diag = lax.rem(d + 2, N_DEV)
        c_own = lax.rem(d + 1, N_DEV)

        barrier_sem = pltpu.get_barrier_semaphore()
        for nbr in [left, right, diag]:
            pl.semaphore_signal(
                barrier_sem, inc=1,
                device_id=(nbr,), device_id_type=pl.DeviceIdType.MESH,
            )
        pl.semaphore_wait(barrier_sem, 3)

        def batch_of(t):
            return lax.rem(d - (t % B) + N_DEV, N_DEV)

        def start_kv_copy(t):
            b = batch_of(t)
            slot = t % 2
            copies = []
            for h in range(H_LOC):
                ck = pltpu.make_async_copy(
                    k_hbm.at[pl.ds(b, 1), :, h, :],
                    kbuf.at[slot, h], ksems.at[slot])
                cv = pltpu.make_async_copy(
                    v_hbm.at[pl.ds(b, 1), :, h, :],
                    vbuf.at[slot, h], vsems.at[slot])
                ck.start()
                cv.start()
                copies.append((ck, cv))
            return copies

        def compute_partial(t, kv):
            p, _ = divmod(t, B)
            b = batch_of(t)
            slot = t % 2
            xb = x_ref[pl.ds(b, 1), pl.ds(p * R, R)].reshape(R, D)
            qb = jnp.dot(xb, wq_ref[...],
                         preferred_element_type=jnp.float32)
            for ck, cv in kv:
                ck.wait()
                cv.wait()
            for h in range(H_LOC):
                qh = qb[:, h * DH:(h + 1) * DH]
                kh = kbuf[slot, h, 0]
                vh = vbuf[slot, h, 0]
                s = lax.dot_general(
                    qh, kh, (((1,), (1,)), ((), ())),
                    preferred_element_type=jnp.float32) * SCALE
                pe = jnp.exp(s)
                l = jnp.sum(pe, axis=-1, keepdims=True)
                o = jnp.dot(pe, vh, preferred_element_type=jnp.float32) / l
                attn_ref[:, h * DH:(h + 1) * DH] = o
            part_ref[t % 2] = jnp.dot(
                attn_ref[...], wo_ref[...],
                preferred_element_type=jnp.float32)

        def rs_desc(t):
            p, j = divmod(t, B)
            tgt = [left, diag, right][j]
            ch = j
            return pltpu.make_async_remote_copy(
                src_ref=part_ref.at[t % 2],
                dst_ref=rs_in.at[p, ch],
                send_sem=rs_send.at[p * 3 + ch],
                recv_sem=rs_recv.at[p * 3 + ch],
                device_id=(tgt,),
                device_id_type=pl.DeviceIdType.MESH,
            )

        def ag_desc(p, ch, chunk, tgt):
            sl = (pl.ds(chunk, 1), pl.ds(p * R, R))
            return pltpu.make_async_remote_copy(
                src_ref=out_ref.at[sl],
                dst_ref=out_ref.at[sl],
                send_sem=ag_send.at[p * 3 + ch],
                recv_sem=ag_recv.at[p * 3 + ch],
                device_id=(tgt,),
                device_id_type=pl.DeviceIdType.MESH,
            )

        all_sends = []

        def finish_phase(p):
            for ch in range(3):
                pltpu.make_async_remote_copy(
                    src_ref=rs_in.at[p, ch],
                    dst_ref=rs_in.at[p, ch],
                    send_sem=rs_send.at[p * 3 + ch],
                    recv_sem=rs_recv.at[p * 3 + ch],
                    device_id=(right,),
                    device_id_type=pl.DeviceIdType.MESH,
                ).wait_recv()
            total = (part_ref[(p * B + 3) % 2]
                     + rs_in[p, 0] + rs_in[p, 1] + rs_in[p, 2])
            out_ref[pl.ds(c_own, 1), pl.ds(p * R, R)] = total.reshape(1, R, D)
            for ch, tgt in [(0, right), (1, left), (2, diag)]:
                s = ag_desc(p, ch, c_own, tgt)
                s.start()
                all_sends.append(s)

        def wait_ag_recvs(p):
            pairs = [
                (0, d),
                (1, lax.rem(d + 2, N_DEV)),
                (2, lax.rem(d + 3, N_DEV)),
            ]
            for ch, chunk in pairs:
                ag_desc(p, ch, chunk, right).wait_recv()

        rs_descs = {}
        kv = start_kv_copy(0)
        for t in range(N_SLOT):
            kv_next = start_kv_copy(t + 1) if t + 1 < N_SLOT else None
            if t - 2 in rs_descs:
                rs_descs[t - 2].wait_send()
            compute_partial(t, kv)
            kv = kv_next
            if t % B < 3:
                r = rs_desc(t)
                r.start()
                rs_descs[t] = r
            if t == B:
                finish_phase(0)
            if t == B + 2:
                wait_ag_recvs(0)
        finish_phase(1)
        wait_ag_recvs(1)

        for t, r in rs_descs.items():
            if t >= N_SLOT - 2:
                r.wait_send()
        for s in all_sends:
            s.wait_send()

    return pl.pallas_call(
        body,
        out_shape=jax.ShapeDtypeStruct((B, SQ, D), jnp.float32),
        in_specs=[
            pl.BlockSpec(memory_space=pltpu.VMEM),
            pl.BlockSpec(memory_space=pltpu.VMEM),
            pl.BlockSpec(memory_space=pltpu.VMEM),
            pl.BlockSpec(memory_space=pl.ANY),
            pl.BlockSpec(memory_space=pl.ANY),
        ],
        out_specs=pl.BlockSpec(memory_space=pltpu.VMEM),
        scratch_shapes=[
            pltpu.VMEM((2, R, D), jnp.float32),
            pltpu.VMEM((P, 3, R, D), jnp.float32),
            pltpu.VMEM((R, H_LOC * DH), jnp.float32),
            pltpu.VMEM((2, H_LOC, 1, SKV, DH), jnp.float32),
            pltpu.VMEM((2, H_LOC, 1, SKV, DH), jnp.float32),
            pltpu.SemaphoreType.DMA((2,)),
            pltpu.SemaphoreType.DMA((2,)),
            pltpu.SemaphoreType.DMA((P * 3,)),
            pltpu.SemaphoreType.DMA((P * 3,)),
            pltpu.SemaphoreType.DMA((P * 3,)),
            pltpu.SemaphoreType.DMA((P * 3,)),
        ],
        compiler_params=pltpu.CompilerParams(
            collective_id=0,
            vmem_limit_bytes=56 * 1024 * 1024,
        ),
    )(x, Wq, Wo, K_ext, V_ext)


# device time: 71779 ns/iter; 1.1122x vs baseline; 1.1122x over previous
import jax
import jax.numpy as jnp
from jax import lax
from jax.experimental import pallas as pl
from jax.experimental.pallas import tpu as pltpu

N_DEV = 4
B = 4
SQ = 256
SKV = 1024
H_LOC = 8
DH = 128
D = 1024
SCALE = 0.08838834764831843

P = 1
R = SQ // P
N_SLOT = P * B


def kernel(x, Wq, Wo, K_ext, V_ext):
    def body(x_ref, wq_ref, wo_ref, k_hbm, v_hbm, out_ref,
             part_ref, rs_in, attn_ref, kbuf, vbuf,
             ksems, vsems, rs_send, rs_recv, ag_send, ag_recv):
        d = lax.axis_index("i")
        left = lax.rem(d + N_DEV - 1, N_DEV)
        right = lax.rem(d + 1, N_DEV)
        diag = lax.rem(d + 2, N_DEV)
        c_own = lax.rem(d + 1, N_DEV)

        barrier_sem = pltpu.get_barrier_semaphore()
        for nbr in [left, right, diag]:
            pl.semaphore_signal(
                barrier_sem, inc=1,
                device_id=(nbr,), device_id_type=pl.DeviceIdType.MESH,
            )
        pl.semaphore_wait(barrier_sem, 3)

        def batch_of(t):
            return lax.rem(d - (t % B) + N_DEV, N_DEV)

        def start_kv_copy(t):
            b = batch_of(t)
            slot = t % 2
            copies = []
            for h in range(H_LOC):
                ck = pltpu.make_async_copy(
                    k_hbm.at[pl.ds(b, 1), :, h, :],
                    kbuf.at[slot, h], ksems.at[slot])
                cv = pltpu.make_async_copy(
                    v_hbm.at[pl.ds(b, 1), :, h, :],
                    vbuf.at[slot, h], vsems.at[slot])
                ck.start()
                cv.start()
                copies.append((ck, cv))
            return copies

        def compute_partial(t, kv):
            p, _ = divmod(t, B)
            b = batch_of(t)
            slot = t % 2
            xb = x_ref[pl.ds(b, 1), pl.ds(p * R, R)].reshape(R, D)
            qb = jnp.dot(xb, wq_ref[...],
                         preferred_element_type=jnp.float32)
            for ck, cv in kv:
                ck.wait()
                cv.wait()
            for h in range(H_LOC):
                qh = qb[:, h * DH:(h + 1) * DH]
                kh = kbuf[slot, h, 0]
                vh = vbuf[slot, h, 0]
                s = lax.dot_general(
                    qh, kh, (((1,), (1,)), ((), ())),
                    preferred_element_type=jnp.float32) * SCALE
                pe = jnp.exp(s)
                l = jnp.sum(pe, axis=-1, keepdims=True)
                o = jnp.dot(pe, vh, preferred_element_type=jnp.float32) / l
                attn_ref[:, h * DH:(h + 1) * DH] = o
            part_ref[t % 2] = jnp.dot(
                attn_ref[...], wo_ref[...],
                preferred_element_type=jnp.float32)

        def rs_desc(t):
            p, j = divmod(t, B)
            tgt = [left, diag, right][j]
            ch = j
            return pltpu.make_async_remote_copy(
                src_ref=part_ref.at[t % 2],
                dst_ref=rs_in.at[p, ch],
                send_sem=rs_send.at[p * 3 + ch],
                recv_sem=rs_recv.at[p * 3 + ch],
                device_id=(tgt,),
                device_id_type=pl.DeviceIdType.MESH,
            )

        def ag_desc(p, ch, chunk, tgt, off=0, rows=R):
            sl = (pl.ds(chunk, 1), pl.ds(p * R + off, rows))
            return pltpu.make_async_remote_copy(
                src_ref=out_ref.at[sl],
                dst_ref=out_ref.at[sl],
                send_sem=ag_send.at[p * 4 + ch],
                recv_sem=ag_recv.at[p * 4 + ch],
                device_id=(tgt,),
                device_id_type=pl.DeviceIdType.MESH,
            )

        all_sends = []

        def finish_phase(p):
            for ch in range(3):
                pltpu.make_async_remote_copy(
                    src_ref=rs_in.at[p, ch],
                    dst_ref=rs_in.at[p, ch],
                    send_sem=rs_send.at[p * 3 + ch],
                    recv_sem=rs_recv.at[p * 3 + ch],
                    device_id=(right,),
                    device_id_type=pl.DeviceIdType.MESH,
                ).wait_recv()
            total = (part_ref[(p * B + 3) % 2]
                     + rs_in[p, 0] + rs_in[p, 1] + rs_in[p, 2])
            out_ref[pl.ds(c_own, 1), pl.ds(p * R, R)] = total.reshape(1, R, D)
            for ch, tgt in [(0, right), (1, left)]:
                s = ag_desc(p, ch, c_own, tgt)
                s.start()
                all_sends.append(s)

        def wait_ag_recvs(p):
            HF = R // 2
            c_from_l = d
            c_from_r = lax.rem(d + 2, N_DEV)
            c_diag = lax.rem(d + 3, N_DEV)
            ag_desc(p, 0, c_from_l, right).wait_recv()
            f_r = ag_desc(p, 2, c_from_l, right, off=0, rows=HF)
            f_r.start()
            all_sends.append(f_r)
            ag_desc(p, 1, c_from_r, left).wait_recv()
            f_l = ag_desc(p, 3, c_from_r, left, off=HF, rows=HF)
            f_l.start()
            all_sends.append(f_l)
            ag_desc(p, 2, c_diag, right, off=0, rows=HF).wait_recv()
            ag_desc(p, 3, c_diag, left, off=HF, rows=HF).wait_recv()

        rs_descs = {}
        kv = start_kv_copy(0)
        for t in range(N_SLOT):
            kv_next = start_kv_copy(t + 1) if t + 1 < N_SLOT else None
            if t - 2 in rs_descs:
                rs_descs[t - 2].wait_send()
            compute_partial(t, kv)
            kv = kv_next
            if t % B < 3:
                r = rs_desc(t)
                r.start()
                rs_descs[t] = r
            if t % B == 0 and t // B >= 1:
                finish_phase(t // B - 1)
            if t % B == 2 and t // B >= 1:
                wait_ag_recvs(t // B - 1)
        finish_phase(P - 1)
        wait_ag_recvs(P - 1)

        for t, r in rs_descs.items():
            if t >= N_SLOT - 2:
                r.wait_send()
        for s in all_sends:
            s.wait_send()

    return pl.pallas_call(
        body,
        out_shape=jax.ShapeDtypeStruct((B, SQ, D), jnp.float32),
        in_specs=[
            pl.BlockSpec(memory_space=pltpu.VMEM),
            pl.BlockSpec(memory_space=pltpu.VMEM),
            pl.BlockSpec(memory_space=pltpu.VMEM),
            pl.BlockSpec(memory_space=pl.ANY),
            pl.BlockSpec(memory_space=pl.ANY),
        ],
        out_specs=pl.BlockSpec(memory_space=pltpu.VMEM),
        scratch_shapes=[
            pltpu.VMEM((2, R, D), jnp.float32),
            pltpu.VMEM((P, 3, R, D), jnp.float32),
            pltpu.VMEM((R, H_LOC * DH), jnp.float32),
            pltpu.VMEM((2, H_LOC, 1, SKV, DH), jnp.float32),
            pltpu.VMEM((2, H_LOC, 1, SKV, DH), jnp.float32),
            pltpu.SemaphoreType.DMA((2,)),
            pltpu.SemaphoreType.DMA((2,)),
            pltpu.SemaphoreType.DMA((P * 3,)),
            pltpu.SemaphoreType.DMA((P * 3,)),
            pltpu.SemaphoreType.DMA((P * 4,)),
            pltpu.SemaphoreType.DMA((P * 4,)),
        ],
        compiler_params=pltpu.CompilerParams(
            collective_id=0,
            vmem_limit_bytes=56 * 1024 * 1024,
        ),
    )(x, Wq, Wo, K_ext, V_ext)
